# Initial kernel scaffold; baseline (speedup 1.0000x reference)
import jax
import jax.numpy as jnp
from jax.experimental import pallas as pl


def kernel(x_graph_1, edge_index_graph_1, batch_graph_1, pre0_W_graph_1, pre0_b_graph_1, pre0_g_graph_1, pre0_be_graph_1, pre1_W_graph_1, pre1_b_graph_1, pre1_g_graph_1, pre1_be_graph_1, conv0_W_graph_1, conv0_b_graph_1, conv1_W_graph_1, conv1_b_graph_1, post0_W_graph_1, post0_b_graph_1, post0_g_graph_1, post0_be_graph_1, post1_W_graph_1, post1_b_graph_1, post1_g_graph_1, post1_be_graph_1, x_graph_2, edge_index_graph_2, batch_graph_2, pre0_W_graph_2, pre0_b_graph_2, pre0_g_graph_2, pre0_be_graph_2, pre1_W_graph_2, pre1_b_graph_2, pre1_g_graph_2, pre1_be_graph_2, conv0_W_graph_2, conv0_b_graph_2, conv1_W_graph_2, conv1_b_graph_2, post0_W_graph_2, post0_b_graph_2, post0_g_graph_2, post0_be_graph_2, post1_W_graph_2, post1_b_graph_2, post1_g_graph_2, post1_be_graph_2, fin0_0_W, fin0_0_b, fin0_1_W, fin0_1_b, fin0_2_W, fin0_2_b, fin1_0_W, fin1_0_b, fin1_1_W, fin1_1_b, fin1_2_W, fin1_2_b):
    raise NotImplementedError("write your pallas kernel here")


# speedup vs baseline: 5.4089x; 5.4089x over previous
import functools

import jax
import jax.numpy as jnp
from jax import lax
from jax.experimental import pallas as pl
from jax.experimental.pallas import tpu as pltpu
from jax.experimental.pallas import tpu_sc as plsc

N = 10000
E = 320000
D = 128
H = 128
OUT = 2
B = 16
NC = 2
NS = 16
L = 16
NPAD = 10112
RPT = NPAD // NS

_MESH = plsc.VectorSubcoreMesh(
    core_axis_name="c", subcore_axis_name="s", num_cores=NC, num_subcores=NS)


CW = 128
CPW = 80
HCH = CPW // 2
EPAD = NC * NS * CPW * CW


@functools.partial(
    pl.kernel,
    out_type=jax.ShapeDtypeStruct((NC * NPAD, H), jnp.float32),
    mesh=_MESH,
    scratch_types=[
        pltpu.VMEM((HCH, CW), jnp.int32),
        pltpu.VMEM((HCH, CW), jnp.int32),
        pltpu.VMEM((CW, H), jnp.float32),
        pltpu.VMEM((CW, H), jnp.float32),
        pltpu.VMEM_SHARED((NPAD, H), jnp.float32),
        pltpu.SemaphoreType.DMA,
        pltpu.SemaphoreType.DMA,
    ],
)
def _sc_aggregate(srcc, dstc, ys, zeros_hbm, out,
                  src_all, dst_all, rows0, rows1, acc, sem0, sem1):
    cid = lax.axis_index("c")
    sid = lax.axis_index("s")
    wid = sid * NC + cid
    pltpu.sync_copy(zeros_hbm, acc.at[pl.ds(sid * RPT, RPT)])
    plsc.subcore_barrier()

    for h in range(CPW // HCH):
        pltpu.sync_copy(srcc.at[wid, pl.ds(h * HCH, HCH)], src_all)
        pltpu.sync_copy(dstc.at[wid, pl.ds(h * HCH, HCH)], dst_all)
        pltpu.async_copy(ys.at[src_all.at[0]], rows0, sem0)

        def body(it, carry):
            i = it * 2
            pltpu.async_copy(ys.at[src_all.at[i + 1]], rows1, sem1)
            pltpu.make_async_copy(ys.at[src_all.at[i]], rows0, sem0).wait()
            pltpu.sync_copy(rows0, acc.at[dst_all.at[i]], add=True)

            @pl.when(it < HCH // 2 - 1)
            def _():
                pltpu.async_copy(ys.at[src_all.at[i + 2]], rows0, sem0)

            pltpu.make_async_copy(ys.at[src_all.at[i + 1]], rows1, sem1).wait()
            pltpu.sync_copy(rows1, acc.at[dst_all.at[i + 1]], add=True)
            return carry

        lax.fori_loop(0, HCH // 2, body, 0)

    plsc.subcore_barrier()
    pltpu.sync_copy(acc.at[pl.ds(sid * RPT, RPT)],
                    out.at[pl.ds(cid * NPAD + sid * RPT, RPT)])


_TC_PARAMS = pltpu.CompilerParams(vmem_limit_bytes=100 * 1024 * 1024)


def _dot(a, b):
    return jnp.dot(a, b, preferred_element_type=jnp.float32,
                   precision=lax.Precision.HIGHEST)


def _bn(h, g, be):
    mu = jnp.mean(h, axis=0, keepdims=True)
    var = jnp.mean((h - mu) ** 2, axis=0, keepdims=True)
    return (h - mu) * lax.rsqrt(var + 1e-5) * g + be


def _dinv_of(dega_ref, degb_ref):
    return lax.rsqrt(dega_ref[...][:, 0:1] + degb_ref[...][:, 0:1] + 1.0)


def _tc_pre(x, w, b, g, be, cw, dega, degb):
    def body(x_ref, w_ref, b_ref, g_ref, be_ref, cw_ref, dega_ref, degb_ref,
             x0_ref, ys_ref):
        h = _dot(x_ref[...], w_ref[...]) + b_ref[...]
        x0 = jnp.maximum(_bn(h, g_ref[...], be_ref[...]), 0.0)
        x0_ref[...] = x0
        ys_ref[...] = _dot(x0, cw_ref[...]) * _dinv_of(dega_ref, degb_ref)

    return pl.pallas_call(
        body,
        out_shape=(jax.ShapeDtypeStruct((N, H), jnp.float32),) * 2,
        compiler_params=_TC_PARAMS,
    )(x, w, b, g, be, cw, dega, degb)


_GRID = 5
_BR = N // _GRID


def _row_specs():
    blk = pl.BlockSpec((_BR, H), lambda g: (g, 0))
    dblk = pl.BlockSpec((_BR, L), lambda g: (g, 0))
    full_h = pl.BlockSpec((1, H), lambda g: (0, 0))
    full_w = pl.BlockSpec((H, H), lambda g: (0, 0))
    return blk, dblk, full_h, full_w


def _tc_mid(x0, ys, agg_a, agg_b, dega, degb, b0, w1):
    def body(x0_ref, ys_ref, agga_ref, aggb_ref, dega_ref, degb_ref, b0_ref,
             w1_ref, x1_ref, ys1_ref):
        dinv = _dinv_of(dega_ref, degb_ref)
        gcn = dinv * (agga_ref[...] + aggb_ref[...] + ys_ref[...]) + b0_ref[...]
        x1 = jnp.maximum(x0_ref[...] + gcn, 0.0)
        x1_ref[...] = x1
        ys1_ref[...] = _dot(x1, w1_ref[...]) * dinv

    blk, dblk, full_h, full_w = _row_specs()
    return pl.pallas_call(
        body,
        grid=(_GRID,),
        in_specs=[blk, blk, blk, blk, dblk, dblk, full_h, full_w],
        out_specs=(blk, blk),
        out_shape=(jax.ShapeDtypeStruct((N, H), jnp.float32),) * 2,
        compiler_params=_TC_PARAMS,
    )(x0, ys, agg_a, agg_b, dega, degb, b0, w1)


def _tc_fin1(x1, ys1, agg_a, agg_b, dega, degb, b1, p0w, p0b):
    def body(x1_ref, ys1_ref, agga_ref, aggb_ref, dega_ref, degb_ref, b1_ref,
             p0w_ref, p0b_ref, h0_ref):
        dinv = _dinv_of(dega_ref, degb_ref)
        gcn = dinv * (agga_ref[...] + aggb_ref[...] + ys1_ref[...]) + b1_ref[...]
        x2 = jnp.maximum(x1_ref[...] + gcn, 0.0)
        h0_ref[...] = _dot(x2, p0w_ref[...]) + p0b_ref[...]

    blk, dblk, full_h, full_w = _row_specs()
    return pl.pallas_call(
        body,
        grid=(_GRID,),
        in_specs=[blk, blk, blk, blk, dblk, dblk, full_h, full_w, full_h],
        out_specs=blk,
        out_shape=jax.ShapeDtypeStruct((N, H), jnp.float32),
        compiler_params=_TC_PARAMS,
    )(x1, ys1, agg_a, agg_b, dega, degb, b1, p0w, p0b)


def _tc_fin2(h0, p0g, p0be, p1w, p1b, p1g, p1be,
             batch, f0w, f0b, f1w, f1b, f2w, f2b):
    def body(h0_ref, p0g_ref, p0be_ref,
             p1w_ref, p1b_ref, p1g_ref, p1be_ref,
             batch_ref, f0w_ref, f0b_ref, f1w_ref, f1b_ref, f2w_ref, f2b_ref,
             out_ref):
        p = jnp.maximum(_bn(h0_ref[...], p0g_ref[...], p0be_ref[...]), 0.0)
        q = _bn(_dot(p, p1w_ref[...]) + p1b_ref[...], p1g_ref[...], p1be_ref[...])
        seg = lax.broadcasted_iota(jnp.int32, (B, N), 0)
        m = (seg == batch_ref[...]).astype(jnp.float32)
        s = _dot(m, q)
        cnt = jnp.sum(m, axis=1, keepdims=True)
        rep = s / jnp.maximum(cnt, 1.0)
        rep = jnp.maximum(_dot(rep, f0w_ref[...]) + f0b_ref[...], 0.0)
        rep = jnp.maximum(_dot(rep, f1w_ref[...]) + f1b_ref[...], 0.0)
        out_ref[...] = _dot(rep, f2w_ref[...]) + f2b_ref[...]

    return pl.pallas_call(
        body,
        out_shape=jax.ShapeDtypeStruct((B, OUT), jnp.float32),
        compiler_params=_TC_PARAMS,
    )(h0, p0g, p0be, p1w, p1b, p1g, p1be,
      batch, f0w, f0b, f1w, f1b, f2w, f2b)


def kernel(x_graph_1, edge_index_graph_1, batch_graph_1, pre0_W_graph_1, pre0_b_graph_1, pre0_g_graph_1, pre0_be_graph_1, pre1_W_graph_1, pre1_b_graph_1, pre1_g_graph_1, pre1_be_graph_1, conv0_W_graph_1, conv0_b_graph_1, conv1_W_graph_1, conv1_b_graph_1, post0_W_graph_1, post0_b_graph_1, post0_g_graph_1, post0_be_graph_1, post1_W_graph_1, post1_b_graph_1, post1_g_graph_1, post1_be_graph_1, x_graph_2, edge_index_graph_2, batch_graph_2, pre0_W_graph_2, pre0_b_graph_2, pre0_g_graph_2, pre0_be_graph_2, pre1_W_graph_2, pre1_b_graph_2, pre1_g_graph_2, pre1_be_graph_2, conv0_W_graph_2, conv0_b_graph_2, conv1_W_graph_2, conv1_b_graph_2, post0_W_graph_2, post0_b_graph_2, post0_g_graph_2, post0_be_graph_2, post1_W_graph_2, post1_b_graph_2, post1_g_graph_2, post1_be_graph_2, fin0_0_W, fin0_0_b, fin0_1_W, fin0_1_b, fin0_2_W, fin0_2_b, fin1_0_W, fin1_0_b, fin1_1_W, fin1_1_b, fin1_2_W, fin1_2_b):
    r1 = lambda a: a.reshape(1, -1)

    pad_src = jnp.zeros((EPAD - E,), jnp.int32)
    pad_dst = jnp.full((EPAD - E,), NPAD - 1, jnp.int32)

    def chunked(col, pad):
        return jnp.concatenate([col.astype(jnp.int32), pad]).reshape(
            NC * NS, CPW, CW)

    src1 = chunked(edge_index_graph_1[0], pad_src)
    dst1 = chunked(edge_index_graph_1[1], pad_dst)
    src2 = chunked(edge_index_graph_2[0], pad_src)
    dst2 = chunked(edge_index_graph_2[1], pad_dst)
    zeros_nh = jnp.zeros((RPT, H), jnp.float32)
    ones_nh = jnp.ones((N, H), jnp.float32)

    def agg_parts(srcc, dstc, ys):
        part = _sc_aggregate(srcc, dstc, ys, zeros_nh)
        return part[:N], part[NPAD:NPAD + N]

    dg1a, dg1b = agg_parts(src1, dst1, ones_nh)
    dg2a, dg2b = agg_parts(src2, dst2, ones_nh)
    dg1a, dg1b = dg1a[:, :L], dg1b[:, :L]
    dg2a, dg2b = dg2a[:, :L], dg2b[:, :L]

    x0_1, ys0_1 = _tc_pre(x_graph_1, pre1_W_graph_1, r1(pre1_b_graph_1),
                          r1(pre1_g_graph_1), r1(pre1_be_graph_1),
                          conv0_W_graph_1, dg1a, dg1b)
    x0_2, ys0_2 = _tc_pre(x_graph_2, pre1_W_graph_2, r1(pre1_b_graph_2),
                          r1(pre1_g_graph_2), r1(pre1_be_graph_2),
                          conv0_W_graph_2, dg2a, dg2b)

    a0_1a, a0_1b = agg_parts(src1, dst1, ys0_1)
    a0_2a, a0_2b = agg_parts(src2, dst2, ys0_2)

    x1_1, ys1_1 = _tc_mid(x0_1, ys0_1, a0_1a, a0_1b, dg1a, dg1b,
                          r1(conv0_b_graph_1), conv1_W_graph_1)
    x1_2, ys1_2 = _tc_mid(x0_2, ys0_2, a0_2a, a0_2b, dg2a, dg2b,
                          r1(conv0_b_graph_2), conv1_W_graph_2)

    a1_1a, a1_1b = agg_parts(src1, dst1, ys1_1)
    a1_2a, a1_2b = agg_parts(src2, dst2, ys1_2)

    h0_1 = _tc_fin1(x1_1, ys1_1, a1_1a, a1_1b, dg1a, dg1b,
                    r1(conv1_b_graph_1), post0_W_graph_1, r1(post0_b_graph_1))
    h0_2 = _tc_fin1(x1_2, ys1_2, a1_2a, a1_2b, dg2a, dg2b,
                    r1(conv1_b_graph_2), post0_W_graph_2, r1(post0_b_graph_2))

    o1 = _tc_fin2(h0_1, r1(post0_g_graph_1), r1(post0_be_graph_1),
                  post1_W_graph_1, r1(post1_b_graph_1), r1(post1_g_graph_1),
                  r1(post1_be_graph_1),
                  batch_graph_1.astype(jnp.int32).reshape(1, N),
                  fin0_0_W, r1(fin0_0_b), fin0_1_W, r1(fin0_1_b),
                  fin0_2_W, r1(fin0_2_b))
    o2 = _tc_fin2(h0_2, r1(post0_g_graph_2), r1(post0_be_graph_2),
                  post1_W_graph_2, r1(post1_b_graph_2), r1(post1_g_graph_2),
                  r1(post1_be_graph_2),
                  batch_graph_2.astype(jnp.int32).reshape(1, N),
                  fin1_0_W, r1(fin1_0_b), fin1_1_W, r1(fin1_1_b),
                  fin1_2_W, r1(fin1_2_b))
    return o1 + o2
